# baseline (speedup 1.0000x reference)
import functools

import jax
import jax.numpy as jnp
from jax import lax
from jax.experimental import pallas as pl
from jax.experimental.pallas import tpu as pltpu
from jax.experimental.pallas import tpu_sc as plsc

_ROW_OFF = (0, 2, 15, 46)
_COL_OFF = (0, 1, 7, 19)
_DIMS = (1, 6, 12, 10)
_TOT_ROWS = 70
_EMB = 29
_F = 20
_OUT_F = 49


def _make_sc_call(n, bc):
    info = plsc.get_sparse_core_info()
    nc, ns = info.num_cores, info.num_subcores
    nw = nc * ns
    n_per_w = n // nw
    n_chunks = n_per_w // bc

    mesh = plsc.VectorSubcoreMesh(core_axis_name="c", subcore_axis_name="s")

    @functools.partial(
        pl.kernel,
        mesh=mesh,
        out_type=jax.ShapeDtypeStruct((n, _OUT_F), jnp.float32),
        compiler_params=pltpu.CompilerParams(needs_layout_passes=False),
        scratch_types=[
            pltpu.VMEM((_TOT_ROWS, _EMB), jnp.float32),
            pltpu.VMEM((bc, _F), jnp.float32),
            pltpu.VMEM((bc, _OUT_F), jnp.float32),
        ],
    )
    def sc_call(x_hbm, t_hbm, out_hbm, t_v, in_v, out_v):
        wid = lax.axis_index("s") * nc + lax.axis_index("c")
        base0 = wid * n_per_w
        pltpu.sync_copy(t_hbm, t_v)

        def chunk_body(ci, carry):
            base = base0 + ci * bc
            pltpu.sync_copy(x_hbm.at[pl.ds(base, bc)], in_v)

            def group_body(g, carry2):
                r16 = lax.iota(jnp.int32, 16) + g * 16
                idx = []
                for k in range(4):
                    col = jnp.full((16,), 16 + k, jnp.int32)
                    v = plsc.load_gather(in_v, [r16, col])
                    idx.append(v.astype(jnp.int32) + _ROW_OFF[k])
                for c in range(_F):
                    colv = jnp.full((16,), c, jnp.int32)
                    vals = plsc.load_gather(in_v, [r16, colv])
                    plsc.store_scatter(out_v, [r16, colv], vals)
                for k in range(4):
                    for j in range(_DIMS[k]):
                        tcol = jnp.full((16,), _COL_OFF[k] + j, jnp.int32)
                        vals = plsc.load_gather(t_v, [idx[k], tcol])
                        ocol = jnp.full((16,), _F + _COL_OFF[k] + j, jnp.int32)
                        plsc.store_scatter(out_v, [r16, ocol], vals)
                return carry2

            lax.fori_loop(0, bc // 16, group_body, 0, unroll=False)
            pltpu.sync_copy(out_v, out_hbm.at[pl.ds(base, bc)])
            return carry

        lax.fori_loop(0, n_chunks, chunk_body, 0, unroll=False)

    return sc_call


def kernel(data, year_table, month_table, day_table, hour_table):
    b, l, f = data.shape
    n = b * l

    t = jnp.zeros((_TOT_ROWS, _EMB), jnp.float32)
    for tab, ro, co in zip(
        (year_table, month_table, day_table, hour_table), _ROW_OFF, _COL_OFF):
        t = lax.dynamic_update_slice(t, tab, (ro, co))

    x2 = data.reshape(n, f)
    out = _make_sc_call(n, 256)(x2, t)
    return out.reshape(b, l, _OUT_F)


# speedup vs baseline: 1.0875x; 1.0875x over previous
import functools

import jax
import jax.numpy as jnp
from jax import lax
from jax.experimental import pallas as pl
from jax.experimental.pallas import tpu as pltpu
from jax.experimental.pallas import tpu_sc as plsc

_F = 20
_OUT_F = 49
_NCOMB = 2 * 13 * 31 * 24


def _make_sc_call(n, bc):
    info = plsc.get_sparse_core_info()
    nc, ns = info.num_cores, info.num_subcores
    nw = nc * ns
    n_per_w = n // nw
    n_chunks = n_per_w // bc

    mesh = plsc.VectorSubcoreMesh(core_axis_name="c", subcore_axis_name="s")

    @functools.partial(
        pl.kernel,
        mesh=mesh,
        out_type=jax.ShapeDtypeStruct((n, _OUT_F), jnp.float32),
        compiler_params=pltpu.CompilerParams(needs_layout_passes=False, use_tc_tiling_on_sc=False),
        scratch_types=[
            pltpu.VMEM((bc, _F), jnp.float32),
            pltpu.VMEM((bc, _OUT_F), jnp.float32),
            pltpu.VMEM((bc,), jnp.int32),
            pltpu.SemaphoreType.DMA,
        ],
    )
    def sc_call(x_hbm, ct_hbm, out_hbm, in_v, out_v, idx_v, sem):
        wid = lax.axis_index("s") * nc + lax.axis_index("c")
        base0 = wid * n_per_w

        def chunk_body(ci, carry):
            base = base0 + ci * bc
            pltpu.sync_copy(x_hbm.at[pl.ds(base, bc)], in_v)

            def group_body(g, carry2):
                r16 = lax.iota(jnp.int32, 16) + g * 16
                f = []
                for k in range(4):
                    col = jnp.full((16,), 16 + k, jnp.int32)
                    f.append(plsc.load_gather(in_v, [r16, col]).astype(jnp.int32))
                comb = ((f[0] * 13 + f[1]) * 31 + f[2]) * 24 + f[3]
                idx_v[pl.ds(g * 16, 16)] = comb
                return carry2

            lax.fori_loop(0, bc // 16, group_body, 0, unroll=False)
            pltpu.async_copy(ct_hbm.at[idx_v], out_v, sem).wait()

            def stitch_body(g, carry2):
                r16 = lax.iota(jnp.int32, 16) + g * 16
                for c in range(_F):
                    colv = jnp.full((16,), c, jnp.int32)
                    vals = plsc.load_gather(in_v, [r16, colv])
                    plsc.store_scatter(out_v, [r16, colv], vals)
                return carry2

            lax.fori_loop(0, bc // 16, stitch_body, 0, unroll=False)
            pltpu.sync_copy(out_v, out_hbm.at[pl.ds(base, bc)])
            return carry

        lax.fori_loop(0, n_chunks, chunk_body, 0, unroll=False)

    return sc_call


def kernel(data, year_table, month_table, day_table, hour_table):
    b, l, f = data.shape
    n = b * l

    ny, nm, nd, nh = 2, 13, 31, 24
    ct = jnp.concatenate(
        [
            jnp.zeros((ny, nm, nd, nh, _F), jnp.float32),
            jnp.broadcast_to(year_table[:, None, None, None, :], (ny, nm, nd, nh, 1)),
            jnp.broadcast_to(month_table[None, :, None, None, :], (ny, nm, nd, nh, 6)),
            jnp.broadcast_to(day_table[None, None, :, None, :], (ny, nm, nd, nh, 12)),
            jnp.broadcast_to(hour_table[None, None, None, :, :], (ny, nm, nd, nh, 10)),
        ],
        axis=-1,
    ).reshape(_NCOMB, _OUT_F)

    x2 = data.reshape(n, f)
    out = _make_sc_call(n, 128)(x2, ct)
    return out.reshape(b, l, _OUT_F)


# speedup vs baseline: 1.1936x; 1.0976x over previous
import functools

import jax
import jax.numpy as jnp
from jax import lax
from jax.experimental import pallas as pl
from jax.experimental.pallas import tpu as pltpu
from jax.experimental.pallas import tpu_sc as plsc

_F = 20
_OUT_F = 49
_DIMS = (1, 6, 12, 10)
_FLAT_OFF = (0, 2, 80, 452)
_FLAT_LEN = 692


def _make_sc_call(n, bc):
    info = plsc.get_sparse_core_info()
    nc, ns = info.num_cores, info.num_subcores
    nw = nc * ns
    n_per_w = n // nw
    n_chunks = n_per_w // bc

    mesh = plsc.VectorSubcoreMesh(core_axis_name="c", subcore_axis_name="s")

    @functools.partial(
        pl.kernel,
        mesh=mesh,
        out_type=jax.ShapeDtypeStruct((n, _OUT_F), jnp.float32),
        compiler_params=pltpu.CompilerParams(
            needs_layout_passes=False, use_tc_tiling_on_sc=False),
        scratch_types=[
            pltpu.VMEM((_FLAT_LEN,), jnp.float32),
            pltpu.VMEM((bc, _F), jnp.float32),
            pltpu.VMEM((bc, _OUT_F), jnp.float32),
            pltpu.VMEM((4, bc), jnp.int32),
        ],
    )
    def sc_call(x_hbm, t_hbm, out_hbm, t_v, in_v, out_v, base_v):
        wid = lax.axis_index("s") * nc + lax.axis_index("c")
        base0 = wid * n_per_w
        pltpu.sync_copy(t_hbm, t_v)

        def chunk_body(ci, carry):
            base = base0 + ci * bc
            pltpu.sync_copy(x_hbm.at[pl.ds(base, bc)], in_v)

            @plsc.parallel_loop(0, bc // 16, 1, unroll=4)
            def idx_body(g):
                r16 = lax.iota(jnp.int32, 16) + g * 16
                for k in range(4):
                    col = jnp.full((16,), 16 + k, jnp.int32)
                    v = plsc.load_gather(in_v, [r16, col]).astype(jnp.int32)
                    base_v[k, pl.ds(g * 16, 16)] = _FLAT_OFF[k] + v * _DIMS[k]

            @plsc.parallel_loop(0, bc // 16, 1, unroll=2)
            def emb_body(g):
                r16 = lax.iota(jnp.int32, 16) + g * 16
                for c in range(_F):
                    colv = jnp.full((16,), c, jnp.int32)
                    vals = plsc.load_gather(in_v, [r16, colv])
                    plsc.store_scatter(out_v, [r16, colv], vals)
                c = _F
                for k in range(4):
                    bk = base_v[k, pl.ds(g * 16, 16)]
                    for j in range(_DIMS[k]):
                        vals = plsc.load_gather(t_v, [bk + j])
                        colv = jnp.full((16,), c, jnp.int32)
                        plsc.store_scatter(out_v, [r16, colv], vals)
                        c += 1

            pltpu.sync_copy(out_v, out_hbm.at[pl.ds(base, bc)])
            return carry

        lax.fori_loop(0, n_chunks, chunk_body, 0, unroll=False)

    return sc_call


def kernel(data, year_table, month_table, day_table, hour_table):
    b, l, f = data.shape
    n = b * l
    t = jnp.concatenate([
        year_table.reshape(-1), month_table.reshape(-1),
        day_table.reshape(-1), hour_table.reshape(-1)])
    x2 = data.reshape(n, f)
    out = _make_sc_call(n, 256)(x2, t)
    return out.reshape(b, l, _OUT_F)


# speedup vs baseline: 1.2350x; 1.0347x over previous
import functools

import jax
import jax.numpy as jnp
from jax import lax
from jax.experimental import pallas as pl
from jax.experimental.pallas import tpu as pltpu
from jax.experimental.pallas import tpu_sc as plsc

_F = 20
_OUT_F = 49
_DIMS = (1, 6, 12, 10)
_FLAT_OFF = (0, 2, 80, 452)
_FLAT_LEN = 692


def _make_sc_call(n, bc):
    info = plsc.get_sparse_core_info()
    nc, ns = info.num_cores, info.num_subcores
    nw = nc * ns
    n_per_w = n // nw
    n_chunks = n_per_w // bc

    mesh = plsc.VectorSubcoreMesh(core_axis_name="c", subcore_axis_name="s")

    @functools.partial(
        pl.kernel,
        mesh=mesh,
        out_type=jax.ShapeDtypeStruct((n * _OUT_F,), jnp.float32),
        compiler_params=pltpu.CompilerParams(
            needs_layout_passes=False, use_tc_tiling_on_sc=False),
        scratch_types=[
            pltpu.VMEM((_FLAT_LEN,), jnp.float32),
            pltpu.VMEM((bc * _F,), jnp.float32),
            pltpu.VMEM((bc * _OUT_F,), jnp.float32),
            pltpu.VMEM((4, bc), jnp.int32),
        ],
    )
    def sc_call(x_hbm, t_hbm, out_hbm, t_v, in_v, out_v, base_v):
        wid = lax.axis_index("s") * nc + lax.axis_index("c")
        base0 = wid * n_per_w
        pltpu.sync_copy(t_hbm, t_v)

        def chunk_body(ci, carry):
            base = base0 + ci * bc
            pltpu.sync_copy(x_hbm.at[pl.ds(base * _F, bc * _F)], in_v)

            @plsc.parallel_loop(0, bc // 16, 1, unroll=4)
            def idx_body(g):
                off20 = (lax.iota(jnp.int32, 16) + g * 16) * _F
                for k in range(4):
                    v = plsc.load_gather(in_v, [off20 + (16 + k)]).astype(jnp.int32)
                    base_v[k, pl.ds(g * 16, 16)] = _FLAT_OFF[k] + v * _DIMS[k]

            @plsc.parallel_loop(0, bc // 16, 1, unroll=2)
            def emb_body(g):
                r16 = lax.iota(jnp.int32, 16) + g * 16
                off20 = r16 * _F
                off49 = r16 * _OUT_F
                for c in range(_F):
                    vals = plsc.load_gather(in_v, [off20 + c])
                    plsc.store_scatter(out_v, [off49 + c], vals)
                c = _F
                for k in range(4):
                    bk = base_v[k, pl.ds(g * 16, 16)]
                    for j in range(_DIMS[k]):
                        vals = plsc.load_gather(t_v, [bk + j])
                        plsc.store_scatter(out_v, [off49 + c], vals)
                        c += 1

            pltpu.sync_copy(out_v, out_hbm.at[pl.ds(base * _OUT_F, bc * _OUT_F)])
            return carry

        lax.fori_loop(0, n_chunks, chunk_body, 0, unroll=False)

    return sc_call


def kernel(data, year_table, month_table, day_table, hour_table):
    b, l, f = data.shape
    n = b * l
    t = jnp.concatenate([
        year_table.reshape(-1), month_table.reshape(-1),
        day_table.reshape(-1), hour_table.reshape(-1)])
    x1 = data.reshape(n * f)
    out = _make_sc_call(n, 256)(x1, t)
    return out.reshape(b, l, _OUT_F)


# speedup vs baseline: 1.4144x; 1.1452x over previous
import functools

import jax
import jax.numpy as jnp
from jax import lax
from jax.experimental import pallas as pl
from jax.experimental.pallas import tpu as pltpu
from jax.experimental.pallas import tpu_sc as plsc

_F = 20
_OUT_F = 49
_DIMS = (1, 6, 12, 10)
_FLAT_OFF = (0, 2, 80, 452)
_FLAT_LEN = 692


def _make_sc_call(n, bc):
    info = plsc.get_sparse_core_info()
    nc, ns = info.num_cores, info.num_subcores
    nw = nc * ns
    n_per_w = n // nw
    n_chunks = n_per_w // bc

    mesh = plsc.VectorSubcoreMesh(core_axis_name="c", subcore_axis_name="s")

    @functools.partial(
        pl.kernel,
        mesh=mesh,
        out_type=jax.ShapeDtypeStruct((n, _OUT_F), jnp.float32),
        compiler_params=pltpu.CompilerParams(
            needs_layout_passes=False, use_tc_tiling_on_sc=True),
        scratch_types=[
            pltpu.VMEM((_FLAT_LEN,), jnp.float32),
            pltpu.VMEM((bc, _F), jnp.float32),
            pltpu.VMEM((bc, _OUT_F), jnp.float32),
            pltpu.VMEM((4, bc), jnp.int32),
        ],
    )
    def sc_call(x_hbm, t_hbm, out_hbm, t_v, in_v, out_v, base_v):
        wid = lax.axis_index("s") * nc + lax.axis_index("c")
        base0 = wid * n_per_w
        pltpu.sync_copy(t_hbm, t_v)

        def chunk_body(ci, carry):
            base = base0 + ci * bc
            pltpu.sync_copy(x_hbm.at[pl.ds(base, bc)], in_v)

            @plsc.parallel_loop(0, bc // 16, 1, unroll=4)
            def idx_body(g):
                r16 = lax.iota(jnp.int32, 16) + g * 16
                for k in range(4):
                    col = jnp.full((16,), 16 + k, jnp.int32)
                    v = plsc.load_gather(in_v, [r16, col]).astype(jnp.int32)
                    base_v[k, pl.ds(g * 16, 16)] = _FLAT_OFF[k] + v * _DIMS[k]

            @plsc.parallel_loop(0, bc // 16, 1, unroll=2)
            def emb_body(g):
                r16 = lax.iota(jnp.int32, 16) + g * 16
                for c in range(_F):
                    colv = jnp.full((16,), c, jnp.int32)
                    vals = plsc.load_gather(in_v, [r16, colv])
                    plsc.store_scatter(out_v, [r16, colv], vals)
                c = _F
                for k in range(4):
                    bk = base_v[k, pl.ds(g * 16, 16)]
                    for j in range(_DIMS[k]):
                        vals = plsc.load_gather(t_v, [bk + j])
                        colv = jnp.full((16,), c, jnp.int32)
                        plsc.store_scatter(out_v, [r16, colv], vals)
                        c += 1

            pltpu.sync_copy(out_v, out_hbm.at[pl.ds(base, bc)])
            return carry

        lax.fori_loop(0, n_chunks, chunk_body, 0, unroll=False)

    return sc_call


def kernel(data, year_table, month_table, day_table, hour_table):
    b, l, f = data.shape
    n = b * l
    t = jnp.concatenate([
        year_table.reshape(-1), month_table.reshape(-1),
        day_table.reshape(-1), hour_table.reshape(-1)])
    x2 = data.reshape(n, f)
    out = _make_sc_call(n, 256)(x2, t)
    return out.reshape(b, l, _OUT_F)
